# Initial kernel scaffold; baseline (speedup 1.0000x reference)
import jax
import jax.numpy as jnp
from jax.experimental import pallas as pl


def kernel(features, edge_index, bn_gamma, bn_beta, W, Wl, Wr):
    raise NotImplementedError("write your pallas kernel here")


# speedup vs baseline: 19.0411x; 19.0411x over previous
import functools
import math

import jax
import jax.numpy as jnp
import numpy as np
from jax import lax
from jax.experimental import pallas as pl
from jax.experimental.pallas import tpu as pltpu
from jax.experimental.pallas import tpu_sc as plsc

EPS = 1e-5
NT = 16
LANES = 16
CHUNK = 128


def _tc_body(x_ref, scale_ref, beta_ref, w_ref, wl_ref, wr_ref,
             ft_ref, a1_ref, a2_ref):
    last = x_ref[...] * scale_ref[...] + beta_ref[...]
    a1s, a2s = [], []
    for h in range(w_ref.shape[0]):
        ft = lax.dot_general(last, w_ref[h], (((1,), (1,)), ((), ())),
                             preferred_element_type=jnp.float32)
        ft_ref[h] = ft
        a1s.append(jnp.sum(ft * wl_ref[h], axis=1))
        a2s.append(jnp.sum(ft * wr_ref[h], axis=1))
    a1_ref[...] = jnp.stack(a1s)
    a2_ref[...] = jnp.stack(a2s)


def _tc_project(xp, scale, beta, W, Wl, Wr, npad, nblk):
    heads, out_f, in_f = W.shape
    blk = npad // nblk
    return pl.pallas_call(
        _tc_body,
        grid=(nblk,),
        in_specs=[
            pl.BlockSpec((blk, in_f), lambda i: (i, 0)),
            pl.BlockSpec((1, in_f), lambda i: (0, 0)),
            pl.BlockSpec((1, in_f), lambda i: (0, 0)),
            pl.BlockSpec((heads, out_f, in_f), lambda i: (0, 0, 0)),
            pl.BlockSpec((heads, 1, out_f), lambda i: (0, 0, 0)),
            pl.BlockSpec((heads, 1, out_f), lambda i: (0, 0, 0)),
        ],
        out_specs=[
            pl.BlockSpec((heads, blk, out_f), lambda i: (0, i, 0)),
            pl.BlockSpec((heads, blk), lambda i: (0, i)),
            pl.BlockSpec((heads, blk), lambda i: (0, i)),
        ],
        out_shape=[
            jax.ShapeDtypeStruct((heads, npad, out_f), jnp.float32),
            jax.ShapeDtypeStruct((heads, npad), jnp.float32),
            jax.ShapeDtypeStruct((heads, npad), jnp.float32),
        ],
    )(xp, scale, beta, W, Wl, Wr)


IW = 16


def _make_sc_edge(n, npad, out_f, nchunks):
    nd = n + LANES
    osub = 80
    ncochunks = n // osub
    nsuper = nchunks // IW
    mesh = plsc.VectorSubcoreMesh(core_axis_name="c", subcore_axis_name="s")

    @functools.partial(
        pl.kernel,
        mesh=mesh,
        compiler_params=pltpu.CompilerParams(needs_layout_passes=False),
        out_type=jax.ShapeDtypeStruct((2 * n, out_f), jnp.float32),
        scratch_types=[
            pltpu.VMEM((IW, CHUNK), jnp.int32),
            pltpu.VMEM((IW, CHUNK), jnp.int32),
            pltpu.VMEM((npad,), jnp.float32),
            pltpu.VMEM((npad,), jnp.float32),
            pltpu.VMEM((CHUNK,), jnp.float32),
            pltpu.VMEM((CHUNK, out_f), jnp.float32),
            pltpu.VMEM((osub,), jnp.float32),
            pltpu.VMEM_SHARED((n + 8, out_f), jnp.float32),
            pltpu.VMEM_SHARED((nd,), jnp.float32),
            pltpu.SemaphoreType.DMA,
        ],
    )
    def sc_edge(ft_hbm, a1_hbm, a2_hbm, src_hbm, dst_hbm, zslab_hbm, z1d_hbm,
                out_hbm, srcw, dstw, a1_v, a2_v, exb, rows_v, denw,
                accum, denom, sem):
        h = lax.axis_index("c")
        s = lax.axis_index("s")
        hoff = h * npad

        pltpu.sync_copy(a1_hbm.at[h], a1_v)
        pltpu.sync_copy(a2_hbm.at[h], a2_v)
        for k in range(-(-ncochunks // NT)):
            cid = k * NT + s

            @pl.when(cid < ncochunks)
            def _():
                pltpu.sync_copy(zslab_hbm, accum.at[pl.ds(cid * osub, osub)])

        @pl.when(s == 0)
        def _():
            pltpu.sync_copy(z1d_hbm, denom)

        plsc.subcore_barrier()

        def _mx(ref):
            def body(i, m):
                return jnp.maximum(m, ref[pl.ds(i * LANES, LANES)])
            m = lax.fori_loop(0, npad // LANES, body,
                              jnp.full((LANES,), -3e38, jnp.float32))
            lanes = lax.iota(jnp.int32, LANES)
            for k in (1, 2, 4, 8):
                m = jnp.maximum(
                    m, m.at[lanes ^ k].get(mode="promise_in_bounds"))
            return m

        cmax = _mx(a1_v) + _mx(a2_v)
        cv = jnp.maximum(cmax, 0.01 * cmax)

        def _calc_ex(jj):
            for g in range(CHUNK // LANES):
                sl = pl.ds(g * LANES, LANES)
                a1g = plsc.load_gather(a1_v, [dstw[jj, sl]])
                a2g = plsc.load_gather(a2_v, [srcw[jj, sl]])
                x = a1g + a2g
                exb[sl] = jnp.exp(jnp.maximum(x, 0.01 * x) - cv)

        def pass_a(ks, carry):
            pltpu.sync_copy(src_hbm.at[s, pl.ds(ks * IW, IW)], srcw)
            pltpu.sync_copy(dst_hbm.at[s, pl.ds(ks * IW, IW)], dstw)

            def chunk_a(jj, c2):
                _calc_ex(jj)
                pltpu.sync_copy(exb, denom.at[dstw.at[jj]], add=True)
                return c2

            lax.fori_loop(0, IW, chunk_a, 0)
            return carry

        lax.fori_loop(0, nsuper, pass_a, 0)

        plsc.subcore_barrier()

        def pass_b(ks, carry):
            pltpu.sync_copy(src_hbm.at[s, pl.ds(ks * IW, IW)], srcw)
            pltpu.sync_copy(dst_hbm.at[s, pl.ds(ks * IW, IW)], dstw)

            def chunk_b(jj, c2):
                _calc_ex(jj)
                for g in range(CHUNK // LANES):
                    sl = pl.ds(g * LANES, LANES)
                    srcw[jj, sl] = srcw[jj, sl] + jnp.broadcast_to(
                        hoff, (LANES,))
                pltpu.async_copy(ft_hbm.at[srcw.at[jj]], rows_v, sem).wait()

                def scale_group(g, c3):
                    ev = exb[pl.ds(g * LANES, LANES)]
                    for r in range(LANES):
                        eb = jnp.broadcast_to(ev[r], (LANES,))
                        row = g * LANES + r
                        for v in range(out_f // LANES):
                            sl = pl.ds(v * LANES, LANES)
                            rows_v[row, sl] = rows_v[row, sl] * eb
                    return c3

                lax.fori_loop(0, CHUNK // LANES, scale_group, 0)
                pltpu.sync_copy(rows_v, accum.at[dstw.at[jj]], add=True)
                return c2

            lax.fori_loop(0, IW, chunk_b, 0)
            return carry

        lax.fori_loop(0, nsuper, pass_b, 0)

        plsc.subcore_barrier()

        def copy_out(k, carry):
            cid = k * NT + s

            @pl.when(cid < ncochunks)
            def _():
                pltpu.sync_copy(accum.at[pl.ds(cid * osub, osub)],
                                rows_v.at[pl.ds(0, osub)])
                pltpu.sync_copy(denom.at[pl.ds(cid * osub, osub)], denw)

                def norm_group(g, c3):
                    dv = denw[pl.ds(g * LANES, LANES)]
                    rv = 1.0 / jnp.maximum(dv, 1e-16)
                    for r in range(LANES):
                        db = jnp.broadcast_to(rv[r], (LANES,))
                        row = g * LANES + r
                        for v in range(out_f // LANES):
                            sl = pl.ds(v * LANES, LANES)
                            rows_v[row, sl] = jnp.maximum(
                                rows_v[row, sl] * db, 0.0)
                    return c3

                lax.fori_loop(0, osub // LANES, norm_group, 0)
                pltpu.sync_copy(rows_v.at[pl.ds(0, osub)],
                                out_hbm.at[pl.ds(h * n + cid * osub, osub)])

            return carry

        lax.fori_loop(0, -(-ncochunks // NT), copy_out, 0)

    return sc_edge


def kernel(features, edge_index, bn_gamma, bn_beta, W, Wl, Wr):
    n, in_f = features.shape
    heads, out_f, _ = W.shape
    e = edge_index.shape[1]
    assert heads == 2 and n % NT == 0 and out_f % LANES == 0

    blk = 1024
    nblk = -(-n // blk)
    npad = nblk * blk
    xp = jnp.pad(features, ((0, npad - n), (0, 0)))
    scale = (bn_gamma * (1.0 / np.sqrt(1.0 + EPS))).reshape(1, in_f)
    beta = bn_beta.reshape(1, in_f)

    ft, a1, a2 = _tc_project(xp, scale, beta, W, Wl, Wr, npad, nblk)

    ept = -(-e // NT)
    nchunks = -(-ept // (CHUNK * IW)) * IW
    tot = NT * nchunks * CHUNK
    src = jnp.pad(edge_index[0], (0, tot - e)).reshape(NT, nchunks, CHUNK)
    dst = jnp.pad(edge_index[1], (0, tot - e),
                  constant_values=n).reshape(NT, nchunks, CHUNK)
    zslab = jnp.zeros((80, out_f), jnp.float32)
    z1d = jnp.zeros((n + LANES,), jnp.float32)

    sc_edge = _make_sc_edge(n, npad, out_f, nchunks)
    out_flat = sc_edge(ft.reshape(heads * npad, out_f), a1, a2, src, dst,
                       zslab, z1d)
    return out_flat.reshape(heads, n, out_f).transpose(1, 0, 2).reshape(
        n, heads * out_f)


# speedup vs baseline: 19.9798x; 1.0493x over previous
import functools
import math

import jax
import jax.numpy as jnp
import numpy as np
from jax import lax
from jax.experimental import pallas as pl
from jax.experimental.pallas import tpu as pltpu
from jax.experimental.pallas import tpu_sc as plsc

EPS = 1e-5
NT = 16
LANES = 16
CHUNK = 128


def _tc_body(x_ref, scale_ref, beta_ref, w_ref, wl_ref, wr_ref,
             ft_ref, a1_ref, a2_ref):
    last = x_ref[...] * scale_ref[...] + beta_ref[...]
    a1s, a2s = [], []
    for h in range(w_ref.shape[0]):
        ft = lax.dot_general(last, w_ref[h], (((1,), (1,)), ((), ())),
                             preferred_element_type=jnp.float32)
        ft_ref[h] = ft
        a1s.append(jnp.sum(ft * wl_ref[h], axis=1))
        a2s.append(jnp.sum(ft * wr_ref[h], axis=1))
    a1_ref[...] = jnp.stack(a1s)
    a2_ref[...] = jnp.stack(a2s)


def _tc_project(xp, scale, beta, W, Wl, Wr, npad, nblk):
    heads, out_f, in_f = W.shape
    blk = npad // nblk
    return pl.pallas_call(
        _tc_body,
        grid=(nblk,),
        in_specs=[
            pl.BlockSpec((blk, in_f), lambda i: (i, 0)),
            pl.BlockSpec((1, in_f), lambda i: (0, 0)),
            pl.BlockSpec((1, in_f), lambda i: (0, 0)),
            pl.BlockSpec((heads, out_f, in_f), lambda i: (0, 0, 0)),
            pl.BlockSpec((heads, 1, out_f), lambda i: (0, 0, 0)),
            pl.BlockSpec((heads, 1, out_f), lambda i: (0, 0, 0)),
        ],
        out_specs=[
            pl.BlockSpec((heads, blk, out_f), lambda i: (0, i, 0)),
            pl.BlockSpec((heads, blk), lambda i: (0, i)),
            pl.BlockSpec((heads, blk), lambda i: (0, i)),
        ],
        out_shape=[
            jax.ShapeDtypeStruct((heads, npad, out_f), jnp.float32),
            jax.ShapeDtypeStruct((heads, npad), jnp.float32),
            jax.ShapeDtypeStruct((heads, npad), jnp.float32),
        ],
    )(xp, scale, beta, W, Wl, Wr)


IW = 16


def _make_sc_edge(n, npad, out_f, nchunks):
    nd = n + LANES
    osub = 80
    ncochunks = n // osub
    nsuper = nchunks // IW
    mesh = plsc.VectorSubcoreMesh(core_axis_name="c", subcore_axis_name="s")

    @functools.partial(
        pl.kernel,
        mesh=mesh,
        compiler_params=pltpu.CompilerParams(needs_layout_passes=False),
        out_type=[
            jax.ShapeDtypeStruct((2 * n, out_f), jnp.float32),
            jax.ShapeDtypeStruct((2, NT, nchunks, CHUNK), jnp.float32),
        ],
        scratch_types=[
            pltpu.VMEM((IW, CHUNK), jnp.int32),
            pltpu.VMEM((IW, CHUNK), jnp.int32),
            pltpu.VMEM((IW, CHUNK), jnp.float32),
            pltpu.VMEM((osub,), jnp.float32),
            pltpu.VMEM_SHARED((n + 8, out_f), jnp.float32),
            pltpu.VMEM_SHARED((nd,), jnp.float32),
            pltpu.SemaphoreType.DMA,
            pltpu.SemaphoreType.DMA,
        ],
    )
    def sc_edge(ft_hbm, a1_hbm, a2_hbm, src_hbm, dst_hbm, zslab_hbm, z1d_hbm,
                out_hbm, ex_hbm, srcw, dstw, exw, denw, accum, denom,
                sem0, sem1):
        h = lax.axis_index("c")
        s = lax.axis_index("s")
        hoff = h * npad

        def zinit(k, carry):
            cid = k * NT + s

            @pl.when(cid < ncochunks)
            def _():
                pltpu.sync_copy(zslab_hbm, accum.at[pl.ds(cid * osub, osub)])

            return carry

        lax.fori_loop(0, -(-ncochunks // NT), zinit, 0)

        @pl.when(s == 0)
        def _():
            pltpu.sync_copy(z1d_hbm, denom)

        def phase_a(a1_v, a2_v):
            pltpu.sync_copy(a1_hbm.at[h], a1_v)
            pltpu.sync_copy(a2_hbm.at[h], a2_v)
            plsc.subcore_barrier()

            def _mx(ref):
                def body(i, m):
                    return jnp.maximum(m, ref[pl.ds(i * LANES, LANES)])
                m = lax.fori_loop(0, npad // LANES, body,
                                  jnp.full((LANES,), -3e38, jnp.float32))
                lanes = lax.iota(jnp.int32, LANES)
                for k in (1, 2, 4, 8):
                    m = jnp.maximum(
                        m, m.at[lanes ^ k].get(mode="promise_in_bounds"))
                return m

            cmax = _mx(a1_v) + _mx(a2_v)
            cv = jnp.maximum(cmax, 0.01 * cmax)

            def pass_a(ks, carry):
                pltpu.sync_copy(src_hbm.at[s, pl.ds(ks * IW, IW)], srcw)
                pltpu.sync_copy(dst_hbm.at[s, pl.ds(ks * IW, IW)], dstw)

                def chunk_a(jj, c2):
                    for g in range(CHUNK // LANES):
                        sl = pl.ds(g * LANES, LANES)
                        a1g = plsc.load_gather(a1_v, [dstw[jj, sl]])
                        a2g = plsc.load_gather(a2_v, [srcw[jj, sl]])
                        x = a1g + a2g
                        exw[jj, sl] = jnp.exp(jnp.maximum(x, 0.01 * x) - cv)
                    pltpu.sync_copy(exw.at[jj], denom.at[dstw.at[jj]],
                                    add=True)
                    return c2

                lax.fori_loop(0, IW, chunk_a, 0)
                pltpu.sync_copy(exw, ex_hbm.at[h, s, pl.ds(ks * IW, IW)])
                return carry

            lax.fori_loop(0, nsuper, pass_a, 0)

        pl.run_scoped(phase_a,
                      pltpu.VMEM((npad,), jnp.float32),
                      pltpu.VMEM((npad,), jnp.float32))

        plsc.subcore_barrier()

        def phase_b(rows0, rows1):
            def scale(rows, jj):
                def scale_group(g, c3):
                    ev = exw[jj, pl.ds(g * LANES, LANES)]
                    for r in range(LANES):
                        eb = jnp.broadcast_to(ev[r], (LANES,))
                        row = g * LANES + r
                        for v in range(out_f // LANES):
                            sl = pl.ds(v * LANES, LANES)
                            rows[row, sl] = rows[row, sl] * eb
                    return c3

                lax.fori_loop(0, CHUNK // LANES, scale_group, 0)

            def pass_b(ks, carry):
                pltpu.sync_copy(src_hbm.at[s, pl.ds(ks * IW, IW)], srcw)
                pltpu.sync_copy(dst_hbm.at[s, pl.ds(ks * IW, IW)], dstw)
                pltpu.sync_copy(ex_hbm.at[h, s, pl.ds(ks * IW, IW)], exw)

                def off(jj, c2):
                    for g in range(CHUNK // LANES):
                        sl = pl.ds(g * LANES, LANES)
                        srcw[jj, sl] = srcw[jj, sl] + jnp.broadcast_to(
                            hoff, (LANES,))
                    return c2

                lax.fori_loop(0, IW, off, 0)

                def pairs(jp, c2):
                    j0 = 2 * jp
                    j1 = 2 * jp + 1
                    cp0 = pltpu.async_copy(ft_hbm.at[srcw.at[j0]], rows0,
                                           sem0)
                    cp1 = pltpu.async_copy(ft_hbm.at[srcw.at[j1]], rows1,
                                           sem1)
                    cp0.wait()
                    scale(rows0, j0)
                    pltpu.sync_copy(rows0, accum.at[dstw.at[j0]], add=True)
                    cp1.wait()
                    scale(rows1, j1)
                    pltpu.sync_copy(rows1, accum.at[dstw.at[j1]], add=True)
                    return c2

                lax.fori_loop(0, IW // 2, pairs, 0)
                return carry

            lax.fori_loop(0, nsuper, pass_b, 0)

        pl.run_scoped(phase_b,
                      pltpu.VMEM((CHUNK, out_f), jnp.float32),
                      pltpu.VMEM((CHUNK, out_f), jnp.float32))

        plsc.subcore_barrier()

        def phase_c(rowsc):
            def copy_out(k, carry):
                cid = k * NT + s

                @pl.when(cid < ncochunks)
                def _():
                    pltpu.sync_copy(accum.at[pl.ds(cid * osub, osub)], rowsc)
                    pltpu.sync_copy(denom.at[pl.ds(cid * osub, osub)], denw)

                    def norm_group(g, c3):
                        dv = denw[pl.ds(g * LANES, LANES)]
                        rv = 1.0 / jnp.maximum(dv, 1e-16)
                        for r in range(LANES):
                            db = jnp.broadcast_to(rv[r], (LANES,))
                            row = g * LANES + r
                            for v in range(out_f // LANES):
                                sl = pl.ds(v * LANES, LANES)
                                rowsc[row, sl] = jnp.maximum(
                                    rowsc[row, sl] * db, 0.0)
                        return c3

                    lax.fori_loop(0, osub // LANES, norm_group, 0)
                    pltpu.sync_copy(
                        rowsc, out_hbm.at[pl.ds(h * n + cid * osub, osub)])

                return carry

            lax.fori_loop(0, -(-ncochunks // NT), copy_out, 0)

        pl.run_scoped(phase_c, pltpu.VMEM((osub, out_f), jnp.float32))

    return sc_edge


def kernel(features, edge_index, bn_gamma, bn_beta, W, Wl, Wr):
    n, in_f = features.shape
    heads, out_f, _ = W.shape
    e = edge_index.shape[1]
    assert heads == 2 and n % NT == 0 and out_f % LANES == 0

    blk = 1024
    nblk = -(-n // blk)
    npad = nblk * blk
    xp = jnp.pad(features, ((0, npad - n), (0, 0)))
    scale = (bn_gamma * (1.0 / np.sqrt(1.0 + EPS))).reshape(1, in_f)
    beta = bn_beta.reshape(1, in_f)

    ft, a1, a2 = _tc_project(xp, scale, beta, W, Wl, Wr, npad, nblk)

    ept = -(-e // NT)
    nchunks = -(-ept // (CHUNK * IW)) * IW
    tot = NT * nchunks * CHUNK
    src = jnp.pad(edge_index[0], (0, tot - e)).reshape(NT, nchunks, CHUNK)
    dst = jnp.pad(edge_index[1], (0, tot - e),
                  constant_values=n).reshape(NT, nchunks, CHUNK)
    zslab = jnp.zeros((80, out_f), jnp.float32)
    z1d = jnp.zeros((n + LANES,), jnp.float32)

    sc_edge = _make_sc_edge(n, npad, out_f, nchunks)
    out_flat, _ = sc_edge(ft.reshape(heads * npad, out_f), a1, a2, src, dst,
                          zslab, z1d)
    return out_flat.reshape(heads, n, out_f).transpose(1, 0, 2).reshape(
        n, heads * out_f)


# speedup vs baseline: 24.9741x; 1.2500x over previous
import functools
import math

import jax
import jax.numpy as jnp
import numpy as np
from jax import lax
from jax.experimental import pallas as pl
from jax.experimental.pallas import tpu as pltpu
from jax.experimental.pallas import tpu_sc as plsc

EPS = 1e-5
NT = 16
LANES = 16
CHUNK = 128


def _tc_body(x_ref, scale_ref, beta_ref, w_ref, wl_ref, wr_ref,
             ft_ref, a1_ref, a2_ref):
    last = x_ref[...] * scale_ref[...] + beta_ref[...]
    a1s, a2s = [], []
    for h in range(w_ref.shape[0]):
        ft = lax.dot_general(last, w_ref[h], (((1,), (1,)), ((), ())),
                             preferred_element_type=jnp.float32)
        ft_ref[h] = ft
        a1s.append(jnp.sum(ft * wl_ref[h], axis=1))
        a2s.append(jnp.sum(ft * wr_ref[h], axis=1))
    a1_ref[...] = jnp.stack(a1s)
    a2_ref[...] = jnp.stack(a2s)


def _tc_project(xp, scale, beta, W, Wl, Wr, npad, nblk):
    heads, out_f, in_f = W.shape
    blk = npad // nblk
    return pl.pallas_call(
        _tc_body,
        grid=(nblk,),
        in_specs=[
            pl.BlockSpec((blk, in_f), lambda i: (i, 0)),
            pl.BlockSpec((1, in_f), lambda i: (0, 0)),
            pl.BlockSpec((1, in_f), lambda i: (0, 0)),
            pl.BlockSpec((heads, out_f, in_f), lambda i: (0, 0, 0)),
            pl.BlockSpec((heads, 1, out_f), lambda i: (0, 0, 0)),
            pl.BlockSpec((heads, 1, out_f), lambda i: (0, 0, 0)),
        ],
        out_specs=[
            pl.BlockSpec((heads, blk, out_f), lambda i: (0, i, 0)),
            pl.BlockSpec((heads, blk), lambda i: (0, i)),
            pl.BlockSpec((heads, blk), lambda i: (0, i)),
        ],
        out_shape=[
            jax.ShapeDtypeStruct((heads, npad, out_f), jnp.float32),
            jax.ShapeDtypeStruct((heads, npad), jnp.float32),
            jax.ShapeDtypeStruct((heads, npad), jnp.float32),
        ],
    )(xp, scale, beta, W, Wl, Wr)


IW = 16


def _make_sc_edge(n, npad, out_f, nchunks):
    nd = n + LANES
    osub = 80
    ncochunks = n // osub
    nsuper = nchunks // IW
    mesh = plsc.VectorSubcoreMesh(core_axis_name="c", subcore_axis_name="s")

    @functools.partial(
        pl.kernel,
        mesh=mesh,
        compiler_params=pltpu.CompilerParams(needs_layout_passes=False),
        out_type=[
            jax.ShapeDtypeStruct((2 * n, out_f), jnp.float32),
            jax.ShapeDtypeStruct((2, NT, nchunks, CHUNK), jnp.float32),
        ],
        scratch_types=[
            pltpu.VMEM((IW, CHUNK), jnp.int32),
            pltpu.VMEM((IW, CHUNK), jnp.int32),
            pltpu.VMEM((IW, CHUNK), jnp.float32),
            pltpu.VMEM((osub,), jnp.float32),
            pltpu.VMEM_SHARED((n + 8, out_f), jnp.float32),
            pltpu.VMEM_SHARED((nd,), jnp.float32),
            pltpu.SemaphoreType.DMA,
            pltpu.SemaphoreType.DMA,
        ],
    )
    def sc_edge(ft_hbm, a1_hbm, a2_hbm, src_hbm, dst_hbm, zslab_hbm, z1d_hbm,
                out_hbm, ex_hbm, srcw, dstw, exw, denw, accum, denom,
                sem0, sem1):
        h = lax.axis_index("c")
        s = lax.axis_index("s")
        hoff = h * npad

        def zinit(k, carry):
            cid = k * NT + s

            @pl.when(cid < ncochunks)
            def _():
                pltpu.sync_copy(zslab_hbm, accum.at[pl.ds(cid * osub, osub)])

            return carry

        lax.fori_loop(0, -(-ncochunks // NT), zinit, 0)

        @pl.when(s == 0)
        def _():
            pltpu.sync_copy(z1d_hbm, denom)

        def phase_a(a1_v, a2_v):
            pltpu.sync_copy(a1_hbm.at[h], a1_v)
            pltpu.sync_copy(a2_hbm.at[h], a2_v)
            plsc.subcore_barrier()

            def _mx(ref):
                def body(i, m):
                    return jnp.maximum(m, ref[pl.ds(i * LANES, LANES)])
                m = lax.fori_loop(0, npad // LANES, body,
                                  jnp.full((LANES,), -3e38, jnp.float32))
                lanes = lax.iota(jnp.int32, LANES)
                for k in (1, 2, 4, 8):
                    m = jnp.maximum(
                        m, m.at[lanes ^ k].get(mode="promise_in_bounds"))
                return m

            cmax = _mx(a1_v) + _mx(a2_v)
            cv = jnp.maximum(cmax, 0.01 * cmax)

            def pass_a(ks, carry):
                pltpu.sync_copy(src_hbm.at[s, pl.ds(ks * IW, IW)], srcw)
                pltpu.sync_copy(dst_hbm.at[s, pl.ds(ks * IW, IW)], dstw)

                def chunk_a(jj, c2):
                    for g in range(CHUNK // LANES):
                        sl = pl.ds(g * LANES, LANES)
                        a1g = plsc.load_gather(a1_v, [dstw[jj, sl]])
                        a2g = plsc.load_gather(a2_v, [srcw[jj, sl]])
                        x = a1g + a2g
                        exw[jj, sl] = jnp.exp(jnp.maximum(x, 0.01 * x) - cv)
                    pltpu.sync_copy(exw.at[jj], denom.at[dstw.at[jj]],
                                    add=True)
                    return c2

                lax.fori_loop(0, IW, chunk_a, 0)
                pltpu.sync_copy(exw, ex_hbm.at[h, s, pl.ds(ks * IW, IW)])
                return carry

            lax.fori_loop(0, nsuper, pass_a, 0)

        pl.run_scoped(phase_a,
                      pltpu.VMEM((npad,), jnp.float32),
                      pltpu.VMEM((npad,), jnp.float32))

        plsc.subcore_barrier()

        def phase_b(rows0, rows1):
            def scale(rows, jj):
                def scale_group(g, c3):
                    ev = exw[jj, pl.ds(g * LANES, LANES)]
                    for r in range(LANES):
                        eb = jnp.broadcast_to(ev[r], (LANES,))
                        row = g * LANES + r
                        for v in range(out_f // LANES):
                            sl = pl.ds(v * LANES, LANES)
                            rows[row, sl] = rows[row, sl] * eb
                    return c3

                lax.fori_loop(0, CHUNK // LANES, scale_group, 0)

            def pass_b(ks, carry):
                pltpu.sync_copy(src_hbm.at[s, pl.ds(ks * IW, IW)], srcw)
                pltpu.sync_copy(dst_hbm.at[s, pl.ds(ks * IW, IW)], dstw)
                pltpu.sync_copy(ex_hbm.at[h, s, pl.ds(ks * IW, IW)], exw)

                def off(jj, c2):
                    rep = (ks * IW + jj + s) % 4
                    base = hoff + rep * (2 * npad)
                    for g in range(CHUNK // LANES):
                        sl = pl.ds(g * LANES, LANES)
                        srcw[jj, sl] = srcw[jj, sl] + jnp.broadcast_to(
                            base, (LANES,))
                    return c2

                lax.fori_loop(0, IW, off, 0)

                def pairs(jp, c2):
                    j0 = 2 * jp
                    j1 = 2 * jp + 1
                    cp0 = pltpu.async_copy(ft_hbm.at[srcw.at[j0]], rows0,
                                           sem0)
                    cp1 = pltpu.async_copy(ft_hbm.at[srcw.at[j1]], rows1,
                                           sem1)
                    cp0.wait()
                    scale(rows0, j0)
                    pltpu.sync_copy(rows0, accum.at[dstw.at[j0]], add=True)
                    cp1.wait()
                    scale(rows1, j1)
                    pltpu.sync_copy(rows1, accum.at[dstw.at[j1]], add=True)
                    return c2

                lax.fori_loop(0, IW // 2, pairs, 0)
                return carry

            lax.fori_loop(0, nsuper, pass_b, 0)

        pl.run_scoped(phase_b,
                      pltpu.VMEM((CHUNK, out_f), jnp.float32),
                      pltpu.VMEM((CHUNK, out_f), jnp.float32))

        plsc.subcore_barrier()

        def phase_c(rowsc):
            def copy_out(k, carry):
                cid = k * NT + s

                @pl.when(cid < ncochunks)
                def _():
                    pltpu.sync_copy(accum.at[pl.ds(cid * osub, osub)], rowsc)
                    pltpu.sync_copy(denom.at[pl.ds(cid * osub, osub)], denw)

                    def norm_group(g, c3):
                        dv = denw[pl.ds(g * LANES, LANES)]
                        rv = 1.0 / jnp.maximum(dv, 1e-16)
                        for r in range(LANES):
                            db = jnp.broadcast_to(rv[r], (LANES,))
                            row = g * LANES + r
                            for v in range(out_f // LANES):
                                sl = pl.ds(v * LANES, LANES)
                                rowsc[row, sl] = jnp.maximum(
                                    rowsc[row, sl] * db, 0.0)
                        return c3

                    lax.fori_loop(0, osub // LANES, norm_group, 0)
                    pltpu.sync_copy(
                        rowsc, out_hbm.at[pl.ds(h * n + cid * osub, osub)])

                return carry

            lax.fori_loop(0, -(-ncochunks // NT), copy_out, 0)

        pl.run_scoped(phase_c, pltpu.VMEM((osub, out_f), jnp.float32))

    return sc_edge


def kernel(features, edge_index, bn_gamma, bn_beta, W, Wl, Wr):
    n, in_f = features.shape
    heads, out_f, _ = W.shape
    e = edge_index.shape[1]
    assert heads == 2 and n % NT == 0 and out_f % LANES == 0

    blk = 1024
    nblk = -(-n // blk)
    npad = nblk * blk
    xp = jnp.pad(features, ((0, npad - n), (0, 0)))
    scale = (bn_gamma * (1.0 / np.sqrt(1.0 + EPS))).reshape(1, in_f)
    beta = bn_beta.reshape(1, in_f)

    ft, a1, a2 = _tc_project(xp, scale, beta, W, Wl, Wr, npad, nblk)

    ept = -(-e // NT)
    nchunks = -(-ept // (CHUNK * IW)) * IW
    tot = NT * nchunks * CHUNK
    src = jnp.pad(edge_index[0], (0, tot - e)).reshape(NT, nchunks, CHUNK)
    dst = jnp.pad(edge_index[1], (0, tot - e),
                  constant_values=n).reshape(NT, nchunks, CHUNK)
    zslab = jnp.zeros((80, out_f), jnp.float32)
    z1d = jnp.zeros((n + LANES,), jnp.float32)

    sc_edge = _make_sc_edge(n, npad, out_f, nchunks)
    ftr = jnp.tile(ft.reshape(heads * npad, out_f), (4, 1))
    out_flat, _ = sc_edge(ftr, a1, a2, src, dst, zslab, z1d)
    return out_flat.reshape(heads, n, out_f).transpose(1, 0, 2).reshape(
        n, heads * out_f)


# speedup vs baseline: 29.9868x; 1.2007x over previous
import functools
import math

import jax
import jax.numpy as jnp
import numpy as np
from jax import lax
from jax.experimental import pallas as pl
from jax.experimental.pallas import tpu as pltpu
from jax.experimental.pallas import tpu_sc as plsc

EPS = 1e-5
NT = 16
LANES = 16
CHUNK = 128


def _tc_body(x_ref, scale_ref, beta_ref, w_ref, wl_ref, wr_ref,
             ft_ref, a1_ref, a2_ref):
    last = x_ref[...] * scale_ref[...] + beta_ref[...]
    a1s, a2s = [], []
    for h in range(w_ref.shape[0]):
        ft = lax.dot_general(last, w_ref[h], (((1,), (1,)), ((), ())),
                             preferred_element_type=jnp.float32)
        ft_ref[h] = ft
        a1s.append(jnp.sum(ft * wl_ref[h], axis=1))
        a2s.append(jnp.sum(ft * wr_ref[h], axis=1))
    a1_ref[...] = jnp.stack(a1s)
    a2_ref[...] = jnp.stack(a2s)


def _tc_project(xp, scale, beta, W, Wl, Wr, npad, nblk):
    heads, out_f, in_f = W.shape
    blk = npad // nblk
    return pl.pallas_call(
        _tc_body,
        grid=(nblk,),
        in_specs=[
            pl.BlockSpec((blk, in_f), lambda i: (i, 0)),
            pl.BlockSpec((1, in_f), lambda i: (0, 0)),
            pl.BlockSpec((1, in_f), lambda i: (0, 0)),
            pl.BlockSpec((heads, out_f, in_f), lambda i: (0, 0, 0)),
            pl.BlockSpec((heads, 1, out_f), lambda i: (0, 0, 0)),
            pl.BlockSpec((heads, 1, out_f), lambda i: (0, 0, 0)),
        ],
        out_specs=[
            pl.BlockSpec((heads, blk, out_f), lambda i: (0, i, 0)),
            pl.BlockSpec((heads, blk), lambda i: (0, i)),
            pl.BlockSpec((heads, blk), lambda i: (0, i)),
        ],
        out_shape=[
            jax.ShapeDtypeStruct((heads, npad, out_f), jnp.float32),
            jax.ShapeDtypeStruct((heads, npad), jnp.float32),
            jax.ShapeDtypeStruct((heads, npad), jnp.float32),
        ],
    )(xp, scale, beta, W, Wl, Wr)


IW = 16


def _make_sc_edge(n, npad, out_f, nchunks, nrep):
    nd = n + LANES
    osub = 80
    ncochunks = n // osub
    nsuper = nchunks // IW
    spt = npad // NT
    mesh = plsc.VectorSubcoreMesh(core_axis_name="c", subcore_axis_name="s")

    @functools.partial(
        pl.kernel,
        mesh=mesh,
        compiler_params=pltpu.CompilerParams(needs_layout_passes=False),
        out_type=jax.ShapeDtypeStruct((2 * n, out_f), jnp.float32),
        scratch_types=[
            pltpu.VMEM((IW, CHUNK), jnp.int32),
            pltpu.VMEM((IW, CHUNK), jnp.int32),
            pltpu.VMEM((IW, CHUNK), jnp.float32),
            pltpu.VMEM((IW, CHUNK), jnp.float32),
            pltpu.VMEM((spt,), jnp.float32),
            pltpu.VMEM((osub,), jnp.float32),
            pltpu.VMEM((CHUNK, out_f), jnp.float32),
            pltpu.VMEM((CHUNK, out_f), jnp.float32),
            pltpu.VMEM_SHARED((n + 8, out_f), jnp.float32),
            pltpu.VMEM_SHARED((nd,), jnp.float32),
            pltpu.VMEM_SHARED((npad,), jnp.float32),
            pltpu.VMEM_SHARED((npad,), jnp.float32),
            pltpu.VMEM_SHARED((2 * NT * LANES,), jnp.float32),
            pltpu.SemaphoreType.DMA,
            pltpu.SemaphoreType.DMA,
            pltpu.SemaphoreType.DMA,
            pltpu.SemaphoreType.DMA,
            pltpu.SemaphoreType.DMA,
            pltpu.SemaphoreType.DMA,
        ],
    )
    def sc_edge(ft_hbm, a1_hbm, a2_hbm, src_hbm, dst_hbm, zslab_hbm, z1d_hbm,
                out_hbm, srcw, dstw, a1w, a2w, asl, denw, rows0, rows1,
                accum, denom, a1sp, a2sp, mxb,
                sem_a, sem_d, sem_g0, sem_g1, sem_s0, sem_s1):
        h = lax.axis_index("c")
        s = lax.axis_index("s")

        def zinit(k, carry):
            cid = k * NT + s

            @pl.when(cid < ncochunks)
            def _():
                pltpu.sync_copy(zslab_hbm, accum.at[pl.ds(cid * osub, osub)])

            return carry

        lax.fori_loop(0, -(-ncochunks // NT), zinit, 0)

        @pl.when(s == 0)
        def _():
            pltpu.sync_copy(z1d_hbm, denom)
            pltpu.sync_copy(a1_hbm.at[h], a1sp)

        @pl.when(s == 1)
        def _():
            pltpu.sync_copy(a2_hbm.at[h], a2sp)

        def _mx(src2d):
            pltpu.sync_copy(src2d.at[h, pl.ds(s * spt, spt)], asl)

            def body(i, m):
                return jnp.maximum(m, asl[pl.ds(i * LANES, LANES)])
            m = lax.fori_loop(0, spt // LANES, body,
                              jnp.full((LANES,), -3e38, jnp.float32))
            lanes = lax.iota(jnp.int32, LANES)
            for k in (1, 2, 4, 8):
                m = jnp.maximum(
                    m, m.at[lanes ^ k].get(mode="promise_in_bounds"))
            return m

        m1l = _mx(a1_hbm)
        m2l = _mx(a2_hbm)
        asl[pl.ds(0, LANES)] = m1l
        asl[pl.ds(LANES, LANES)] = m2l
        pltpu.sync_copy(asl.at[pl.ds(0, 2 * LANES)],
                        mxb.at[pl.ds(2 * LANES * s, 2 * LANES)])
        plsc.subcore_barrier()
        pltpu.sync_copy(mxb, asl.at[pl.ds(0, 2 * NT * LANES)])

        def redm(i, mm):
            m1 = jnp.maximum(mm[0], asl[pl.ds(2 * LANES * i, LANES)])
            m2 = jnp.maximum(mm[1], asl[pl.ds(2 * LANES * i + LANES, LANES)])
            return (m1, m2)

        neg = jnp.full((LANES,), -3e38, jnp.float32)
        m1g, m2g = lax.fori_loop(0, NT, redm, (neg, neg))
        cmax = m1g + m2g
        cv = jnp.maximum(cmax, 0.01 * cmax)

        rows = (rows0, rows1)
        sem_g = (sem_g0, sem_g1)
        sem_s = (sem_s0, sem_s1)

        def gather_rows(jj, b):
            return pltpu.async_copy(ft_hbm.at[srcw.at[jj]], rows[b],
                                    sem_g[b])

        def scale(rb, jj):
            def scale_group(g, c3):
                ev = a1w[jj, pl.ds(g * LANES, LANES)]
                for r in range(LANES):
                    eb = jnp.broadcast_to(ev[r], (LANES,))
                    row = g * LANES + r
                    for v in range(out_f // LANES):
                        sl = pl.ds(v * LANES, LANES)
                        rb[row, sl] = rb[row, sl] * eb
                return c3

            lax.fori_loop(0, CHUNK // LANES, scale_group, 0)

        def main(ks, carry):
            pltpu.sync_copy(src_hbm.at[s, pl.ds(ks * IW, IW)], srcw)
            pltpu.sync_copy(dst_hbm.at[s, pl.ds(ks * IW, IW)], dstw)

            ha = []
            for jj in range(IW):
                ha.append(pltpu.async_copy(a1sp.at[dstw.at[jj]],
                                           a1w.at[jj], sem_a))
                ha.append(pltpu.async_copy(a2sp.at[srcw.at[jj]],
                                           a2w.at[jj], sem_a))
            for cp in ha:
                cp.wait()

            def exc(jj, c2):
                for g in range(CHUNK // LANES):
                    sl = pl.ds(g * LANES, LANES)
                    x = a1w[jj, sl] + a2w[jj, sl]
                    a1w[jj, sl] = jnp.exp(jnp.maximum(x, 0.01 * x) - cv)
                return c2

            lax.fori_loop(0, IW, exc, 0)

            def off(jj, c2):
                rep = (ks * IW + jj + s) % nrep
                base = h * npad + rep * (2 * npad)
                for g in range(CHUNK // LANES):
                    sl = pl.ds(g * LANES, LANES)
                    srcw[jj, sl] = srcw[jj, sl] + jnp.broadcast_to(
                        base, (LANES,))
                return c2

            lax.fori_loop(0, IW, off, 0)

            def dscat(jj, c2):
                pltpu.sync_copy(a1w.at[jj], denom.at[dstw.at[jj]], add=True)
                return c2

            lax.fori_loop(0, IW, dscat, 0)

            hg = {0: gather_rows(0, 0), 1: gather_rows(1, 1)}
            for jj in range(IW):
                b = jj % 2
                hg[jj].wait()
                scale(rows[b], jj)
                pltpu.sync_copy(rows[b], accum.at[dstw.at[jj]], add=True)
                if jj + 2 < IW:
                    hg[jj + 2] = gather_rows(jj + 2, b)
            return carry

        lax.fori_loop(0, nsuper, main, 0)

        plsc.subcore_barrier()

        def copy_out(k, carry):
            cid = k * NT + s

            @pl.when(cid < ncochunks)
            def _():
                pltpu.sync_copy(accum.at[pl.ds(cid * osub, osub)],
                                rows0.at[pl.ds(0, osub)])
                pltpu.sync_copy(denom.at[pl.ds(cid * osub, osub)], denw)

                def norm_group(g, c3):
                    dv = denw[pl.ds(g * LANES, LANES)]
                    rv = 1.0 / jnp.maximum(dv, 1e-16)
                    for r in range(LANES):
                        db = jnp.broadcast_to(rv[r], (LANES,))
                        row = g * LANES + r
                        for v in range(out_f // LANES):
                            sl = pl.ds(v * LANES, LANES)
                            rows0[row, sl] = jnp.maximum(
                                rows0[row, sl] * db, 0.0)
                    return c3

                lax.fori_loop(0, osub // LANES, norm_group, 0)
                pltpu.sync_copy(
                    rows0.at[pl.ds(0, osub)],
                    out_hbm.at[pl.ds(h * n + cid * osub, osub)])

            return carry

        lax.fori_loop(0, -(-ncochunks // NT), copy_out, 0)

    return sc_edge


def kernel(features, edge_index, bn_gamma, bn_beta, W, Wl, Wr):
    n, in_f = features.shape
    heads, out_f, _ = W.shape
    e = edge_index.shape[1]
    assert heads == 2 and n % NT == 0 and out_f % LANES == 0

    blk = 1024
    nblk = -(-n // blk)
    npad = nblk * blk
    xp = jnp.pad(features, ((0, npad - n), (0, 0)))
    scale = (bn_gamma * (1.0 / np.sqrt(1.0 + EPS))).reshape(1, in_f)
    beta = bn_beta.reshape(1, in_f)

    ft, a1, a2 = _tc_project(xp, scale, beta, W, Wl, Wr, npad, nblk)

    ept = -(-e // NT)
    nchunks = -(-ept // (CHUNK * IW)) * IW
    tot = NT * nchunks * CHUNK
    src = jnp.pad(edge_index[0], (0, tot - e)).reshape(NT, nchunks, CHUNK)
    dst = jnp.pad(edge_index[1], (0, tot - e),
                  constant_values=n).reshape(NT, nchunks, CHUNK)
    zslab = jnp.zeros((80, out_f), jnp.float32)
    z1d = jnp.zeros((n + LANES,), jnp.float32)

    sc_edge = _make_sc_edge(n, npad, out_f, nchunks, 4)
    ftr = jnp.tile(ft.reshape(heads * npad, out_f), (4, 1))
    out_flat = sc_edge(ftr, a1, a2, src, dst, zslab, z1d)
    return out_flat.reshape(heads, n, out_f).transpose(1, 0, 2).reshape(
        n, heads * out_f)


# speedup vs baseline: 30.2059x; 1.0073x over previous
import functools
import math

import jax
import jax.numpy as jnp
import numpy as np
from jax import lax
from jax.experimental import pallas as pl
from jax.experimental.pallas import tpu as pltpu
from jax.experimental.pallas import tpu_sc as plsc

EPS = 1e-5
NT = 16
LANES = 16
CHUNK = 128


def _tc_body(x_ref, scale_ref, beta_ref, w_ref, wl_ref, wr_ref,
             ft_ref, a1_ref, a2_ref):
    last = x_ref[...] * scale_ref[...] + beta_ref[...]
    a1s, a2s = [], []
    for h in range(w_ref.shape[0]):
        ft = lax.dot_general(last, w_ref[h], (((1,), (1,)), ((), ())),
                             preferred_element_type=jnp.float32)
        ft_ref[h] = ft
        a1s.append(jnp.sum(ft * wl_ref[h], axis=1))
        a2s.append(jnp.sum(ft * wr_ref[h], axis=1))
    a1_ref[...] = jnp.stack(a1s)
    a2_ref[...] = jnp.stack(a2s)


def _tc_project(xp, scale, beta, W, Wl, Wr, npad, nblk):
    heads, out_f, in_f = W.shape
    blk = npad // nblk
    return pl.pallas_call(
        _tc_body,
        grid=(nblk,),
        in_specs=[
            pl.BlockSpec((blk, in_f), lambda i: (i, 0)),
            pl.BlockSpec((1, in_f), lambda i: (0, 0)),
            pl.BlockSpec((1, in_f), lambda i: (0, 0)),
            pl.BlockSpec((heads, out_f, in_f), lambda i: (0, 0, 0)),
            pl.BlockSpec((heads, 1, out_f), lambda i: (0, 0, 0)),
            pl.BlockSpec((heads, 1, out_f), lambda i: (0, 0, 0)),
        ],
        out_specs=[
            pl.BlockSpec((heads, blk, out_f), lambda i: (0, i, 0)),
            pl.BlockSpec((heads, blk), lambda i: (0, i)),
            pl.BlockSpec((heads, blk), lambda i: (0, i)),
        ],
        out_shape=[
            jax.ShapeDtypeStruct((heads, npad, out_f), jnp.float32),
            jax.ShapeDtypeStruct((heads, npad), jnp.float32),
            jax.ShapeDtypeStruct((heads, npad), jnp.float32),
        ],
    )(xp, scale, beta, W, Wl, Wr)


IW = 16


def _make_sc_edge(n, npad, out_f, nchunks, nrep):
    nd = n + LANES
    osub = 80
    ncochunks = n // osub
    nsuper = nchunks // IW
    spt = npad // NT
    mesh = plsc.VectorSubcoreMesh(core_axis_name="c", subcore_axis_name="s")

    @functools.partial(
        pl.kernel,
        mesh=mesh,
        compiler_params=pltpu.CompilerParams(needs_layout_passes=False),
        out_type=jax.ShapeDtypeStruct((2 * n, out_f), jnp.float32),
        scratch_types=[
            pltpu.VMEM((IW, CHUNK), jnp.int32),
            pltpu.VMEM((IW, CHUNK), jnp.int32),
            pltpu.VMEM((IW, CHUNK), jnp.float32),
            pltpu.VMEM((IW, CHUNK), jnp.float32),
            pltpu.VMEM((spt,), jnp.float32),
            pltpu.VMEM((osub,), jnp.float32),
            pltpu.VMEM((CHUNK, out_f), jnp.float32),
            pltpu.VMEM((CHUNK, out_f), jnp.float32),
            pltpu.VMEM_SHARED((n + 8, out_f), jnp.float32),
            pltpu.VMEM_SHARED((nd,), jnp.float32),
            pltpu.VMEM_SHARED((npad,), jnp.float32),
            pltpu.VMEM_SHARED((npad,), jnp.float32),
            pltpu.VMEM_SHARED((2 * NT * LANES,), jnp.float32),
            pltpu.SemaphoreType.DMA,
            pltpu.SemaphoreType.DMA,
            pltpu.SemaphoreType.DMA,
            pltpu.SemaphoreType.DMA,
            pltpu.SemaphoreType.DMA,
            pltpu.SemaphoreType.DMA,
        ],
    )
    def sc_edge(ft_hbm, a1_hbm, a2_hbm, src_hbm, dst_hbm, zslab_hbm, z1d_hbm,
                out_hbm, srcw, dstw, a1w, a2w, asl, denw, rows0, rows1,
                accum, denom, a1sp, a2sp, mxb,
                sem_a, sem_d, sem_g0, sem_g1, sem_s0, sem_s1):
        h = lax.axis_index("c")
        s = lax.axis_index("s")

        def zinit(k, carry):
            cid = k * NT + s

            @pl.when(cid < ncochunks)
            def _():
                pltpu.sync_copy(zslab_hbm, accum.at[pl.ds(cid * osub, osub)])

            return carry

        lax.fori_loop(0, -(-ncochunks // NT), zinit, 0)

        @pl.when(s == 0)
        def _():
            pltpu.sync_copy(z1d_hbm, denom)
            pltpu.sync_copy(a1_hbm.at[h], a1sp)

        @pl.when(s == 1)
        def _():
            pltpu.sync_copy(a2_hbm.at[h], a2sp)

        def _mx(src2d):
            pltpu.sync_copy(src2d.at[h, pl.ds(s * spt, spt)], asl)

            def body(i, m):
                return jnp.maximum(m, asl[pl.ds(i * LANES, LANES)])
            m = lax.fori_loop(0, spt // LANES, body,
                              jnp.full((LANES,), -3e38, jnp.float32))
            lanes = lax.iota(jnp.int32, LANES)
            for k in (1, 2, 4, 8):
                m = jnp.maximum(
                    m, m.at[lanes ^ k].get(mode="promise_in_bounds"))
            return m

        m1l = _mx(a1_hbm)
        m2l = _mx(a2_hbm)
        asl[pl.ds(0, LANES)] = m1l
        asl[pl.ds(LANES, LANES)] = m2l
        pltpu.sync_copy(asl.at[pl.ds(0, 2 * LANES)],
                        mxb.at[pl.ds(2 * LANES * s, 2 * LANES)])
        plsc.subcore_barrier()
        pltpu.sync_copy(mxb, asl.at[pl.ds(0, 2 * NT * LANES)])

        def redm(i, mm):
            m1 = jnp.maximum(mm[0], asl[pl.ds(2 * LANES * i, LANES)])
            m2 = jnp.maximum(mm[1], asl[pl.ds(2 * LANES * i + LANES, LANES)])
            return (m1, m2)

        neg = jnp.full((LANES,), -3e38, jnp.float32)
        m1g, m2g = lax.fori_loop(0, NT, redm, (neg, neg))
        cmax = m1g + m2g
        cv = jnp.maximum(cmax, 0.01 * cmax)

        rows = (rows0, rows1)
        sem_g = (sem_g0, sem_g1)
        sem_s = (sem_s0, sem_s1)

        def gather_rows(jj, b):
            return pltpu.async_copy(ft_hbm.at[srcw.at[jj]], rows[b],
                                    sem_g[b])

        def scale(rb, jj):
            def scale_group(g, c3):
                ev = a1w[jj, pl.ds(g * LANES, LANES)]
                for r in range(LANES):
                    eb = jnp.broadcast_to(ev[r], (LANES,))
                    row = g * LANES + r
                    for v in range(out_f // LANES):
                        sl = pl.ds(v * LANES, LANES)
                        rb[row, sl] = rb[row, sl] * eb
                return c3

            lax.fori_loop(0, CHUNK // LANES, scale_group, 0)

        def main(ks, carry):
            pltpu.sync_copy(src_hbm.at[s, pl.ds(ks * IW, IW)], srcw)
            pltpu.sync_copy(dst_hbm.at[s, pl.ds(ks * IW, IW)], dstw)

            ha = []
            for jj in range(IW):
                ha.append(pltpu.async_copy(a1sp.at[dstw.at[jj]],
                                           a1w.at[jj], sem_a))
                ha.append(pltpu.async_copy(a2sp.at[srcw.at[jj]],
                                           a2w.at[jj], sem_a))
            for cp in ha:
                cp.wait()

            def exc(jj, c2):
                for g in range(CHUNK // LANES):
                    sl = pl.ds(g * LANES, LANES)
                    x = a1w[jj, sl] + a2w[jj, sl]
                    a1w[jj, sl] = jnp.exp(jnp.maximum(x, 0.01 * x) - cv)
                return c2

            lax.fori_loop(0, IW, exc, 0)

            def off(jj, c2):
                rep = (ks * IW + jj + s) % nrep
                base = h * npad + rep * (2 * npad)
                for g in range(CHUNK // LANES):
                    sl = pl.ds(g * LANES, LANES)
                    srcw[jj, sl] = srcw[jj, sl] + jnp.broadcast_to(
                        base, (LANES,))
                return c2

            lax.fori_loop(0, IW, off, 0)

            def dscat(jj, c2):
                pltpu.sync_copy(a1w.at[jj], denom.at[dstw.at[jj]], add=True)
                return c2

            lax.fori_loop(0, IW, dscat, 0)

            hg = {0: gather_rows(0, 0), 1: gather_rows(1, 1)}
            hs = {}
            for jj in range(IW):
                b = jj % 2
                hg[jj].wait()
                scale(rows[b], jj)
                hs[jj] = pltpu.async_copy(rows[b], accum.at[dstw.at[jj]],
                                          sem_s[b], add=True)
                if jj + 2 < IW:
                    hs[jj].wait()
                    hg[jj + 2] = gather_rows(jj + 2, b)
            hs[IW - 2].wait()
            hs[IW - 1].wait()
            return carry

        lax.fori_loop(0, nsuper, main, 0)

        plsc.subcore_barrier()

        def copy_out(k, carry):
            cid = k * NT + s

            @pl.when(cid < ncochunks)
            def _():
                pltpu.sync_copy(accum.at[pl.ds(cid * osub, osub)],
                                rows0.at[pl.ds(0, osub)])
                pltpu.sync_copy(denom.at[pl.ds(cid * osub, osub)], denw)

                def norm_group(g, c3):
                    dv = denw[pl.ds(g * LANES, LANES)]
                    rv = 1.0 / jnp.maximum(dv, 1e-16)
                    for r in range(LANES):
                        db = jnp.broadcast_to(rv[r], (LANES,))
                        row = g * LANES + r
                        for v in range(out_f // LANES):
                            sl = pl.ds(v * LANES, LANES)
                            rows0[row, sl] = jnp.maximum(
                                rows0[row, sl] * db, 0.0)
                    return c3

                lax.fori_loop(0, osub // LANES, norm_group, 0)
                pltpu.sync_copy(
                    rows0.at[pl.ds(0, osub)],
                    out_hbm.at[pl.ds(h * n + cid * osub, osub)])

            return carry

        lax.fori_loop(0, -(-ncochunks // NT), copy_out, 0)

    return sc_edge


def kernel(features, edge_index, bn_gamma, bn_beta, W, Wl, Wr):
    n, in_f = features.shape
    heads, out_f, _ = W.shape
    e = edge_index.shape[1]
    assert heads == 2 and n % NT == 0 and out_f % LANES == 0

    blk = 1024
    nblk = -(-n // blk)
    npad = nblk * blk
    xp = jnp.pad(features, ((0, npad - n), (0, 0)))
    scale = (bn_gamma * (1.0 / np.sqrt(1.0 + EPS))).reshape(1, in_f)
    beta = bn_beta.reshape(1, in_f)

    ft, a1, a2 = _tc_project(xp, scale, beta, W, Wl, Wr, npad, nblk)

    ept = -(-e // NT)
    nchunks = -(-ept // (CHUNK * IW)) * IW
    tot = NT * nchunks * CHUNK
    src = jnp.pad(edge_index[0], (0, tot - e)).reshape(NT, nchunks, CHUNK)
    dst = jnp.pad(edge_index[1], (0, tot - e),
                  constant_values=n).reshape(NT, nchunks, CHUNK)
    zslab = jnp.zeros((80, out_f), jnp.float32)
    z1d = jnp.zeros((n + LANES,), jnp.float32)

    sc_edge = _make_sc_edge(n, npad, out_f, nchunks, 4)
    ftr = jnp.tile(ft.reshape(heads * npad, out_f), (4, 1))
    out_flat = sc_edge(ftr, a1, a2, src, dst, zslab, z1d)
    return out_flat.reshape(heads, n, out_f).transpose(1, 0, 2).reshape(
        n, heads * out_f)
